# baseline (speedup 1.0000x reference)
import jax
import jax.numpy as jnp
from jax.experimental import pallas as pl
from jax.experimental.pallas import tpu as pltpu

DIM_ = 2048
ROWS_ = 4 * 4096
BLOCK_ROWS_ = 512


def _add_tag_kernel(idx_ref, x_ref, emb_ref, o_ref):
    i = idx_ref[0]
    emb = emb_ref[:, :]
    row_ids = jax.lax.broadcasted_iota(jnp.int32, emb.shape, 0)
    tag = jnp.sum(jnp.where(row_ids == i, emb, 0.0), axis=0, keepdims=True)
    o_ref[:, :] = x_ref[:, :] + tag


def kernel(x, embeddings, modality_id):
    idx = jnp.asarray(modality_id, dtype=jnp.int32).reshape((1,))
    x2 = x.reshape(ROWS_, DIM_)
    grid = ROWS_ // BLOCK_ROWS_
    out = pl.pallas_call(
        _add_tag_kernel,
        grid_spec=pltpu.PrefetchScalarGridSpec(
            num_scalar_prefetch=1,
            grid=(grid,),
            in_specs=[
                pl.BlockSpec((BLOCK_ROWS_, DIM_), lambda g, s_ref: (g, 0)),
                pl.BlockSpec(embeddings.shape, lambda g, s_ref: (0, 0)),
            ],
            out_specs=pl.BlockSpec((BLOCK_ROWS_, DIM_), lambda g, s_ref: (g, 0)),
        ),
        out_shape=jax.ShapeDtypeStruct((ROWS_, DIM_), x.dtype),
    )(idx, x2, embeddings)
    return out.reshape(x.shape)


# speedup vs baseline: 1.0193x; 1.0193x over previous
import jax
import jax.numpy as jnp
from jax.experimental import pallas as pl
from jax.experimental.pallas import tpu as pltpu

DIM_ = 2048
ROWS_ = 4 * 4096
BLOCK_ROWS_ = 1024


def _add_tag_kernel(idx_ref, x_ref, emb_ref, o_ref):
    i = idx_ref[0]
    emb = emb_ref[:, :]
    row_ids = jax.lax.broadcasted_iota(jnp.int32, emb.shape, 0)
    tag = jnp.sum(jnp.where(row_ids == i, emb, 0.0), axis=0, keepdims=True)
    o_ref[:, :] = x_ref[:, :] + tag


def kernel(x, embeddings, modality_id):
    idx = jnp.asarray(modality_id, dtype=jnp.int32).reshape((1,))
    x2 = x.reshape(ROWS_, DIM_)
    grid = ROWS_ // BLOCK_ROWS_
    out = pl.pallas_call(
        _add_tag_kernel,
        grid_spec=pltpu.PrefetchScalarGridSpec(
            num_scalar_prefetch=1,
            grid=(grid,),
            in_specs=[
                pl.BlockSpec((BLOCK_ROWS_, DIM_), lambda g, s_ref: (g, 0)),
                pl.BlockSpec(embeddings.shape, lambda g, s_ref: (0, 0)),
            ],
            out_specs=pl.BlockSpec((BLOCK_ROWS_, DIM_), lambda g, s_ref: (g, 0)),
        ),
        out_shape=jax.ShapeDtypeStruct((ROWS_, DIM_), x.dtype),
    )(idx, x2, embeddings)
    return out.reshape(x.shape)
